# baseline (speedup 1.0000x reference)
import functools

import jax
import jax.numpy as jnp
from jax import lax
from jax.experimental import pallas as pl
from jax.experimental.pallas import tpu as pltpu
from jax.experimental.pallas import tpu_sc as plsc

N = 10000
E = 160000
D = 256
DH = 128
NSUB = 16
NCORE = 2
ET = E // NSUB
CHUNK = 80
SUP = 2000
NSUP = ET // SUP
NCHUNK = SUP // CHUNK
ROWS_PT = 624
ROWS_TAIL = N - ROWS_PT * NSUB

RB = 1000
NRB = N // RB


def _mm_y_body(x_ref, wr_ref, y_ref):
    dn = (((1,), (1,)), ((), ()))
    y_ref[...] = lax.dot_general(x_ref[...], wr_ref[...], dn,
                                 preferred_element_type=jnp.float32)


def _y_stage(x, W_rel):
    return pl.pallas_call(
        _mm_y_body,
        grid=(NRB, NCORE),
        in_specs=[
            pl.BlockSpec((RB, D), lambda b, h: (b, 0)),
            pl.BlockSpec((DH, D), lambda b, h: (h, 0)),
        ],
        out_specs=pl.BlockSpec((RB, DH), lambda b, h: (h * NRB + b, 0)),
        out_shape=jax.ShapeDtypeStruct((NCORE * N, DH), jnp.float32),
    )(x, W_rel)


def _sc_body(y_hbm, src_hbm, dst_hbm, ew_hbm, zero_hbm, agg_hbm,
             src_all, dst_all, ew_all,
             srcv0, srcv1, dstv0, dstv1, rows0, rows1,
             agg_sh, gsem0, gsem1, ssem0, ssem1):
    c = lax.axis_index("c")
    t = lax.axis_index("s")

    pltpu.sync_copy(zero_hbm.at[pl.ds(0, ROWS_PT)],
                    agg_sh.at[pl.ds(t * ROWS_PT, ROWS_PT)])

    @pl.when(t == 0)
    def _zero_tail():
        pltpu.sync_copy(zero_hbm.at[pl.ds(0, ROWS_TAIL)],
                        agg_sh.at[pl.ds(NSUB * ROWS_PT, ROWS_TAIL)])

    plsc.subcore_barrier()

    coff = jnp.full((16,), c * N, dtype=jnp.int32)

    def prep(k, srcv, dstv):
        for i in range(CHUNK // 16):
            sl = pl.ds(i * 16, 16)
            sls = pl.ds(k * CHUNK + i * 16, 16)
            srcv[sl] = src_all[sls] + coff
            dstv[sl] = dst_all[sls]

    def gath_issue(srcv, rows, gsem):
        pltpu.async_copy(y_hbm.at[srcv], rows, gsem)

    def gath_wait(srcv, rows, gsem):
        pltpu.make_async_copy(y_hbm.at[srcv], rows, gsem).wait()

    def scat_issue(rows, dstv, ssem):
        pltpu.async_copy(rows, agg_sh.at[dstv], ssem, add=True)

    def scat_wait(rows, dstv, ssem):
        pltpu.make_async_copy(rows, agg_sh.at[dstv], ssem).wait()

    def mult(k, rows):
        def group_body(g, carry2):
            ewv = ew_all[pl.ds(k * CHUNK + g * 16, 16)]
            e0 = g * 16
            for i in range(16):
                w = jnp.full((16,), ewv[i])
                for j in range(DH // 16):
                    sl = pl.ds(j * 16, 16)
                    rows[e0 + i, sl] = rows[e0 + i, sl] * w
            return carry2

        lax.fori_loop(0, CHUNK // 16, group_body, 0)

    bufs = ((srcv0, dstv0, rows0, gsem0, ssem0),
            (srcv1, dstv1, rows1, gsem1, ssem1))

    def step(k, active, other):
        sv_a, dv_a, rw_a, gs_a, ss_a = active
        sv_o, dv_o, rw_o, gs_o, ss_o = other
        scat_wait(rw_o, dv_o, ss_o)

        @pl.when(k + 1 < NCHUNK)
        def _prefetch():
            prep(k + 1, sv_o, dv_o)
            gath_issue(sv_o, rw_o, gs_o)

        gath_wait(sv_a, rw_a, gs_a)
        mult(k, rw_a)
        scat_issue(rw_a, dv_a, ss_a)

    def super_body(q, carry):
        base = t * ET + q * SUP
        pltpu.sync_copy(src_hbm.at[pl.ds(base, SUP)], src_all)
        pltpu.sync_copy(dst_hbm.at[pl.ds(base, SUP)], dst_all)
        pltpu.sync_copy(ew_hbm.at[pl.ds(base, SUP)], ew_all.at[pl.ds(0, SUP)])

        prep(0, srcv0, dstv0)
        gath_issue(srcv0, rows0, gsem0)
        prep(1, srcv1, dstv1)
        gath_issue(srcv1, rows1, gsem1)
        gath_wait(srcv0, rows0, gsem0)
        mult(0, rows0)
        scat_issue(rows0, dstv0, ssem0)

        def pair_body(kk, carry2):
            step(2 * kk + 1, bufs[1], bufs[0])
            step(2 * kk + 2, bufs[0], bufs[1])
            return carry2

        lax.fori_loop(0, (NCHUNK - 1) // 2, pair_body, 0)
        scat_wait(rows0, dstv0, ssem0)
        return carry

    lax.fori_loop(0, NSUP, super_body, 0)
    plsc.subcore_barrier()

    pltpu.sync_copy(agg_sh.at[pl.ds(t * ROWS_PT, ROWS_PT)],
                    agg_hbm.at[pl.ds(c * N + t * ROWS_PT, ROWS_PT)])

    @pl.when(t == 0)
    def _write_tail():
        pltpu.sync_copy(agg_sh.at[pl.ds(NSUB * ROWS_PT, ROWS_TAIL)],
                        agg_hbm.at[pl.ds(c * N + NSUB * ROWS_PT, ROWS_TAIL)])


def _aggregate_stage(y_split, src, dst, ew, zeros):
    mesh = plsc.VectorSubcoreMesh(core_axis_name="c", subcore_axis_name="s")
    kern = pl.kernel(
        _sc_body,
        out_type=jax.ShapeDtypeStruct((NCORE * N, DH), jnp.float32),
        mesh=mesh,
        scratch_types=[
            pltpu.VMEM((SUP,), jnp.int32),
            pltpu.VMEM((SUP,), jnp.int32),
            pltpu.VMEM((SUP + 16,), jnp.float32),
            pltpu.VMEM((CHUNK,), jnp.int32),
            pltpu.VMEM((CHUNK,), jnp.int32),
            pltpu.VMEM((CHUNK,), jnp.int32),
            pltpu.VMEM((CHUNK,), jnp.int32),
            pltpu.VMEM((CHUNK, DH), jnp.float32),
            pltpu.VMEM((CHUNK, DH), jnp.float32),
            pltpu.VMEM_SHARED((N, DH), jnp.float32),
            pltpu.SemaphoreType.DMA,
            pltpu.SemaphoreType.DMA,
            pltpu.SemaphoreType.DMA,
            pltpu.SemaphoreType.DMA,
        ],
    )
    return kern(y_split, src, dst, ew, zeros)


def _combine_body(agg_ref, x_ref, wo_ref, s_ref, b2_ref, out_ref):
    dn = (((1,), (1,)), ((), ()))
    z = lax.dot_general(x_ref[...], wo_ref[...], dn,
                        preferred_element_type=jnp.float32)
    out_ref[...] = s_ref[...] * (agg_ref[...] + z) + b2_ref[...]


def _combine_stage(agg_split, x, W_root, s2d, b2d):
    return pl.pallas_call(
        _combine_body,
        grid=(NRB, NCORE),
        in_specs=[
            pl.BlockSpec((RB, DH), lambda b, h: (h * NRB + b, 0)),
            pl.BlockSpec((RB, D), lambda b, h: (b, 0)),
            pl.BlockSpec((DH, D), lambda b, h: (h, 0)),
            pl.BlockSpec((1, DH), lambda b, h: (0, h)),
            pl.BlockSpec((1, DH), lambda b, h: (0, h)),
        ],
        out_specs=pl.BlockSpec((RB, DH), lambda b, h: (b, h)),
        out_shape=jax.ShapeDtypeStruct((N, D), jnp.float32),
    )(agg_split, x, W_root, s2d, b2d)


@jax.jit
def kernel(x, edge_index, edge_weight, W_rel, b_rel, W_root, gamma, beta):
    src = edge_index[0]
    dst = edge_index[1]
    s = gamma * (1.0 / jnp.sqrt(1.0 + 1e-5))
    b2 = s * b_rel + beta
    zeros = jnp.zeros((ROWS_PT, DH), jnp.float32)

    y_split = _y_stage(x, W_rel)
    agg_split = _aggregate_stage(y_split, src, dst, edge_weight, zeros)
    out = _combine_stage(agg_split, x, W_root,
                         s.reshape(1, D), b2.reshape(1, D))
    return out


# speedup vs baseline: 1.0077x; 1.0077x over previous
import functools

import jax
import jax.numpy as jnp
from jax import lax
from jax.experimental import pallas as pl
from jax.experimental.pallas import tpu as pltpu
from jax.experimental.pallas import tpu_sc as plsc

N = 10000
E = 160000
D = 256
DH = 128
NSUB = 16
NCORE = 2
ET = E // NSUB
CHUNK = 80
SUP = 2000
NSUP = ET // SUP
NCHUNK = SUP // CHUNK
ROWS_PT = 624
ROWS_TAIL = N - ROWS_PT * NSUB

RB = 1000
NRB = N // RB


def _mm_y_body(x_ref, wr_ref, y_ref):
    dn = (((1,), (1,)), ((), ()))
    y_ref[...] = lax.dot_general(x_ref[...], wr_ref[...], dn,
                                 preferred_element_type=jnp.float32)


def _y_stage(x, W_rel):
    return pl.pallas_call(
        _mm_y_body,
        grid=(NRB, NCORE),
        in_specs=[
            pl.BlockSpec((RB, D), lambda b, h: (b, 0)),
            pl.BlockSpec((DH, D), lambda b, h: (h, 0)),
        ],
        out_specs=pl.BlockSpec((RB, DH), lambda b, h: (h * NRB + b, 0)),
        out_shape=jax.ShapeDtypeStruct((NCORE * N, DH), jnp.float32),
    )(x, W_rel)


def _z_stage(x, W_root):
    return pl.pallas_call(
        _mm_y_body,
        grid=(NRB, NCORE),
        in_specs=[
            pl.BlockSpec((RB, D), lambda b, h: (b, 0)),
            pl.BlockSpec((DH, D), lambda b, h: (h, 0)),
        ],
        out_specs=pl.BlockSpec((RB, DH), lambda b, h: (b, h)),
        out_shape=jax.ShapeDtypeStruct((N, D), jnp.float32),
    )(x, W_root)


def _sc_body(y_hbm, src_hbm, dst_hbm, ew_hbm, zero_hbm, agg_hbm,
             src_all, dst_all, ew_all,
             srcv0, srcv1, dstv0, dstv1, rows0, rows1,
             agg_sh, gsem0, gsem1, ssem0, ssem1):
    c = lax.axis_index("c")
    t = lax.axis_index("s")

    pltpu.sync_copy(zero_hbm.at[pl.ds(0, ROWS_PT)],
                    agg_sh.at[pl.ds(t * ROWS_PT, ROWS_PT)])

    @pl.when(t == 0)
    def _zero_tail():
        pltpu.sync_copy(zero_hbm.at[pl.ds(0, ROWS_TAIL)],
                        agg_sh.at[pl.ds(NSUB * ROWS_PT, ROWS_TAIL)])

    plsc.subcore_barrier()

    coff = jnp.full((16,), c * N, dtype=jnp.int32)

    def prep(k, srcv, dstv):
        for i in range(CHUNK // 16):
            sl = pl.ds(i * 16, 16)
            sls = pl.ds(k * CHUNK + i * 16, 16)
            srcv[sl] = src_all[sls] + coff
            dstv[sl] = dst_all[sls]

    HC = CHUNK // 2

    def gath_issue(srcv, rows, gsem):
        pltpu.async_copy(y_hbm.at[srcv.at[pl.ds(0, HC)]],
                         rows.at[pl.ds(0, HC)], gsem)
        pltpu.async_copy(y_hbm.at[srcv.at[pl.ds(HC, HC)]],
                         rows.at[pl.ds(HC, HC)], gsem)

    def gath_wait(srcv, rows, gsem):
        pltpu.make_async_copy(y_hbm.at[srcv.at[pl.ds(0, HC)]],
                              rows.at[pl.ds(0, HC)], gsem).wait()
        pltpu.make_async_copy(y_hbm.at[srcv.at[pl.ds(HC, HC)]],
                              rows.at[pl.ds(HC, HC)], gsem).wait()

    def scat_issue(rows, dstv, ssem):
        pltpu.async_copy(rows, agg_sh.at[dstv], ssem, add=True)

    def scat_wait(rows, dstv, ssem):
        pltpu.make_async_copy(rows, agg_sh.at[dstv], ssem).wait()

    def mult(k, rows):
        def group_body(g, carry2):
            ewv = ew_all[pl.ds(k * CHUNK + g * 16, 16)]
            e0 = g * 16
            for i in range(16):
                w = jnp.full((16,), ewv[i])
                for j in range(DH // 16):
                    sl = pl.ds(j * 16, 16)
                    rows[e0 + i, sl] = rows[e0 + i, sl] * w
            return carry2

        lax.fori_loop(0, CHUNK // 16, group_body, 0)

    bufs = ((srcv0, dstv0, rows0, gsem0, ssem0),
            (srcv1, dstv1, rows1, gsem1, ssem1))

    def step(k, active, other):
        sv_a, dv_a, rw_a, gs_a, ss_a = active
        sv_o, dv_o, rw_o, gs_o, ss_o = other
        scat_wait(rw_o, dv_o, ss_o)

        @pl.when(k + 1 < NCHUNK)
        def _prefetch():
            prep(k + 1, sv_o, dv_o)
            gath_issue(sv_o, rw_o, gs_o)

        gath_wait(sv_a, rw_a, gs_a)
        mult(k, rw_a)
        scat_issue(rw_a, dv_a, ss_a)

    def super_body(q, carry):
        base = t * ET + q * SUP
        pltpu.sync_copy(src_hbm.at[pl.ds(base, SUP)], src_all)
        pltpu.sync_copy(dst_hbm.at[pl.ds(base, SUP)], dst_all)
        pltpu.sync_copy(ew_hbm.at[pl.ds(base, SUP)], ew_all.at[pl.ds(0, SUP)])

        prep(0, srcv0, dstv0)
        gath_issue(srcv0, rows0, gsem0)
        prep(1, srcv1, dstv1)
        gath_issue(srcv1, rows1, gsem1)
        gath_wait(srcv0, rows0, gsem0)
        mult(0, rows0)
        scat_issue(rows0, dstv0, ssem0)

        def pair_body(kk, carry2):
            step(2 * kk + 1, bufs[1], bufs[0])
            step(2 * kk + 2, bufs[0], bufs[1])
            return carry2

        lax.fori_loop(0, (NCHUNK - 1) // 2, pair_body, 0)
        scat_wait(rows0, dstv0, ssem0)
        return carry

    lax.fori_loop(0, NSUP, super_body, 0)
    plsc.subcore_barrier()

    pltpu.sync_copy(agg_sh.at[pl.ds(t * ROWS_PT, ROWS_PT)],
                    agg_hbm.at[pl.ds(c * N + t * ROWS_PT, ROWS_PT)])

    @pl.when(t == 0)
    def _write_tail():
        pltpu.sync_copy(agg_sh.at[pl.ds(NSUB * ROWS_PT, ROWS_TAIL)],
                        agg_hbm.at[pl.ds(c * N + NSUB * ROWS_PT, ROWS_TAIL)])


def _aggregate_stage(y_split, src, dst, ew, zeros):
    mesh = plsc.VectorSubcoreMesh(core_axis_name="c", subcore_axis_name="s")
    kern = pl.kernel(
        _sc_body,
        out_type=jax.ShapeDtypeStruct((NCORE * N, DH), jnp.float32),
        mesh=mesh,
        scratch_types=[
            pltpu.VMEM((SUP,), jnp.int32),
            pltpu.VMEM((SUP,), jnp.int32),
            pltpu.VMEM((SUP + 16,), jnp.float32),
            pltpu.VMEM((CHUNK,), jnp.int32),
            pltpu.VMEM((CHUNK,), jnp.int32),
            pltpu.VMEM((CHUNK,), jnp.int32),
            pltpu.VMEM((CHUNK,), jnp.int32),
            pltpu.VMEM((CHUNK, DH), jnp.float32),
            pltpu.VMEM((CHUNK, DH), jnp.float32),
            pltpu.VMEM_SHARED((N, DH), jnp.float32),
            pltpu.SemaphoreType.DMA,
            pltpu.SemaphoreType.DMA,
            pltpu.SemaphoreType.DMA,
            pltpu.SemaphoreType.DMA,
        ],
    )
    return kern(y_split, src, dst, ew, zeros)


def _combine_body(agg_ref, z_ref, s_ref, b2_ref, out_ref):
    out_ref[...] = s_ref[...] * (agg_ref[...] + z_ref[...]) + b2_ref[...]


def _combine_stage(agg_split, z, s2d, b2d):
    return pl.pallas_call(
        _combine_body,
        grid=(NRB, NCORE),
        in_specs=[
            pl.BlockSpec((RB, DH), lambda b, h: (h * NRB + b, 0)),
            pl.BlockSpec((RB, DH), lambda b, h: (b, h)),
            pl.BlockSpec((1, DH), lambda b, h: (0, h)),
            pl.BlockSpec((1, DH), lambda b, h: (0, h)),
        ],
        out_specs=pl.BlockSpec((RB, DH), lambda b, h: (b, h)),
        out_shape=jax.ShapeDtypeStruct((N, D), jnp.float32),
    )(agg_split, z, s2d, b2d)


@jax.jit
def kernel(x, edge_index, edge_weight, W_rel, b_rel, W_root, gamma, beta):
    src = edge_index[0]
    dst = edge_index[1]
    s = gamma * (1.0 / jnp.sqrt(1.0 + 1e-5))
    b2 = s * b_rel + beta
    zeros = jnp.zeros((ROWS_PT, DH), jnp.float32)

    y_split = _y_stage(x, W_rel)
    agg_split = _aggregate_stage(y_split, src, dst, edge_weight, zeros)
    z = _z_stage(x, W_root)
    out = _combine_stage(agg_split, z, s.reshape(1, D), b2.reshape(1, D))
    return out


# speedup vs baseline: 1.0470x; 1.0390x over previous
import jax
import jax.numpy as jnp
from jax import lax
from jax.experimental import pallas as pl
from jax.experimental.pallas import tpu as pltpu
from jax.experimental.pallas import tpu_sc as plsc

N = 10000
E = 160000
D = 256
DH = 128
NSUB = 16
NCORE = 2
ET = E // NSUB
SUP = 2000
NSUP = ET // SUP
CHUNK = 128
NMAIN = 15
TAIL = SUP - NMAIN * CHUNK
ROWS_PT = 624
ROWS_TAIL = N - ROWS_PT * NSUB

RB = 1000
NRB = N // RB


def _mm_body(x_ref, w_ref, o_ref):
    dn = (((1,), (1,)), ((), ()))
    o_ref[...] = lax.dot_general(x_ref[...], w_ref[...], dn,
                                 preferred_element_type=jnp.float32)


def _y_stage(x, W_rel):
    return pl.pallas_call(
        _mm_body,
        grid=(NRB, NCORE),
        in_specs=[
            pl.BlockSpec((RB, D), lambda b, h: (b, 0)),
            pl.BlockSpec((DH, D), lambda b, h: (h, 0)),
        ],
        out_specs=pl.BlockSpec((RB, DH), lambda b, h: (h * NRB + b, 0)),
        out_shape=jax.ShapeDtypeStruct((NCORE * N, DH), jnp.float32),
    )(x, W_rel)


def _z_stage(x, W_root):
    return pl.pallas_call(
        _mm_body,
        grid=(NRB, NCORE),
        in_specs=[
            pl.BlockSpec((RB, D), lambda b, h: (b, 0)),
            pl.BlockSpec((DH, D), lambda b, h: (h, 0)),
        ],
        out_specs=pl.BlockSpec((RB, DH), lambda b, h: (b, h)),
        out_shape=jax.ShapeDtypeStruct((N, D), jnp.float32),
    )(x, W_root)


def _sc_body(y_hbm, src_hbm, dst_hbm, ew_hbm, zero_hbm, agg_hbm,
             src_all, dst_all, ew_all,
             srcv0, srcv1, dstv0, dstv1, rows0, rows1,
             srcvt, dstvt, rowst,
             agg_sh, gsem0, gsem1, ssem0, ssem1, gsemt, ssemt):
    c = lax.axis_index("c")
    t = lax.axis_index("s")

    pltpu.sync_copy(zero_hbm.at[pl.ds(0, ROWS_PT)],
                    agg_sh.at[pl.ds(t * ROWS_PT, ROWS_PT)])

    @pl.when(t == 0)
    def _zero_tail():
        pltpu.sync_copy(zero_hbm.at[pl.ds(0, ROWS_TAIL)],
                        agg_sh.at[pl.ds(NSUB * ROWS_PT, ROWS_TAIL)])

    plsc.subcore_barrier()

    coff = jnp.full((16,), c * N, dtype=jnp.int32)

    def prep(base, n, srcv, dstv):
        for i in range(n // 16):
            sl = pl.ds(i * 16, 16)
            sls = pl.ds(base + i * 16, 16)
            srcv[sl] = src_all[sls] + coff
            dstv[sl] = dst_all[sls]

    def gath_issue(srcv, rows, gsem):
        pltpu.async_copy(y_hbm.at[srcv], rows, gsem)

    def gath_wait(srcv, rows, gsem):
        pltpu.make_async_copy(y_hbm.at[srcv], rows, gsem).wait()

    def scat_issue(rows, dstv, ssem):
        pltpu.async_copy(rows, agg_sh.at[dstv], ssem, add=True)

    def scat_wait(rows, dstv, ssem):
        pltpu.make_async_copy(rows, agg_sh.at[dstv], ssem).wait()

    def mult(base, n, rows):
        def group_body(g, carry2):
            ewv = ew_all[pl.ds(base + g * 16, 16)]
            e0 = g * 16
            for i in range(16):
                w = jnp.full((16,), ewv[i])
                for j in range(DH // 16):
                    sl = pl.ds(j * 16, 16)
                    rows[e0 + i, sl] = rows[e0 + i, sl] * w
            return carry2

        lax.fori_loop(0, n // 16, group_body, 0)

    bufs = ((srcv0, dstv0, rows0, gsem0, ssem0),
            (srcv1, dstv1, rows1, gsem1, ssem1))
    tailb = (srcvt, dstvt, rowst, gsemt, ssemt)

    def step(k, active, other):
        sv_a, dv_a, rw_a, gs_a, ss_a = active
        sv_o, dv_o, rw_o, gs_o, ss_o = other
        scat_wait(rw_o, dv_o, ss_o)

        @pl.when(k + 1 < NMAIN)
        def _prefetch():
            prep((k + 1) * CHUNK, CHUNK, sv_o, dv_o)
            gath_issue(sv_o, rw_o, gs_o)

        @pl.when(k + 1 == NMAIN)
        def _prefetch_tail():
            prep(NMAIN * CHUNK, TAIL, srcvt, dstvt)
            gath_issue(srcvt, rowst, gsemt)

        gath_wait(sv_a, rw_a, gs_a)
        mult(k * CHUNK, CHUNK, rw_a)
        scat_issue(rw_a, dv_a, ss_a)

    def super_body(q, carry):
        base = t * ET + q * SUP
        pltpu.sync_copy(src_hbm.at[pl.ds(base, SUP)], src_all)
        pltpu.sync_copy(dst_hbm.at[pl.ds(base, SUP)], dst_all)
        pltpu.sync_copy(ew_hbm.at[pl.ds(base, SUP)], ew_all.at[pl.ds(0, SUP)])

        prep(0, CHUNK, srcv0, dstv0)
        gath_issue(srcv0, rows0, gsem0)
        prep(CHUNK, CHUNK, srcv1, dstv1)
        gath_issue(srcv1, rows1, gsem1)
        gath_wait(srcv0, rows0, gsem0)
        mult(0, CHUNK, rows0)
        scat_issue(rows0, dstv0, ssem0)

        def pair_body(kk, carry2):
            step(2 * kk + 1, bufs[1], bufs[0])
            step(2 * kk + 2, bufs[0], bufs[1])
            return carry2

        lax.fori_loop(0, (NMAIN - 1) // 2, pair_body, 0)

        scat_wait(rows0, dstv0, ssem0)
        gath_wait(srcvt, rowst, gsemt)
        mult(NMAIN * CHUNK, TAIL, rowst)
        scat_issue(rowst, dstvt, ssemt)
        scat_wait(rowst, dstvt, ssemt)
        return carry

    lax.fori_loop(0, NSUP, super_body, 0)
    plsc.subcore_barrier()

    pltpu.sync_copy(agg_sh.at[pl.ds(t * ROWS_PT, ROWS_PT)],
                    agg_hbm.at[pl.ds(c * N + t * ROWS_PT, ROWS_PT)])

    @pl.when(t == 0)
    def _write_tail():
        pltpu.sync_copy(agg_sh.at[pl.ds(NSUB * ROWS_PT, ROWS_TAIL)],
                        agg_hbm.at[pl.ds(c * N + NSUB * ROWS_PT, ROWS_TAIL)])


def _aggregate_stage(y_split, src, dst, ew, zeros):
    mesh = plsc.VectorSubcoreMesh(core_axis_name="c", subcore_axis_name="s")
    kern = pl.kernel(
        _sc_body,
        out_type=jax.ShapeDtypeStruct((NCORE * N, DH), jnp.float32),
        mesh=mesh,
        scratch_types=[
            pltpu.VMEM((SUP,), jnp.int32),
            pltpu.VMEM((SUP,), jnp.int32),
            pltpu.VMEM((SUP + 16,), jnp.float32),
            pltpu.VMEM((CHUNK,), jnp.int32),
            pltpu.VMEM((CHUNK,), jnp.int32),
            pltpu.VMEM((CHUNK,), jnp.int32),
            pltpu.VMEM((CHUNK,), jnp.int32),
            pltpu.VMEM((CHUNK, DH), jnp.float32),
            pltpu.VMEM((CHUNK, DH), jnp.float32),
            pltpu.VMEM((TAIL,), jnp.int32),
            pltpu.VMEM((TAIL,), jnp.int32),
            pltpu.VMEM((TAIL, DH), jnp.float32),
            pltpu.VMEM_SHARED((N, DH), jnp.float32),
            pltpu.SemaphoreType.DMA,
            pltpu.SemaphoreType.DMA,
            pltpu.SemaphoreType.DMA,
            pltpu.SemaphoreType.DMA,
            pltpu.SemaphoreType.DMA,
            pltpu.SemaphoreType.DMA,
        ],
    )
    return kern(y_split, src, dst, ew, zeros)


def _combine_body(agg_ref, z_ref, s_ref, b2_ref, out_ref):
    out_ref[...] = s_ref[...] * (agg_ref[...] + z_ref[...]) + b2_ref[...]


def _combine_stage(agg_split, z, s2d, b2d):
    return pl.pallas_call(
        _combine_body,
        grid=(NRB, NCORE),
        in_specs=[
            pl.BlockSpec((RB, DH), lambda b, h: (h * NRB + b, 0)),
            pl.BlockSpec((RB, DH), lambda b, h: (b, h)),
            pl.BlockSpec((1, DH), lambda b, h: (0, h)),
            pl.BlockSpec((1, DH), lambda b, h: (0, h)),
        ],
        out_specs=pl.BlockSpec((RB, DH), lambda b, h: (b, h)),
        out_shape=jax.ShapeDtypeStruct((N, D), jnp.float32),
    )(agg_split, z, s2d, b2d)


@jax.jit
def kernel(x, edge_index, edge_weight, W_rel, b_rel, W_root, gamma, beta):
    src = edge_index[0]
    dst = edge_index[1]
    s = gamma * (1.0 / jnp.sqrt(1.0 + 1e-5))
    b2 = s * b_rel + beta
    zeros = jnp.zeros((ROWS_PT, DH), jnp.float32)

    y_split = _y_stage(x, W_rel)
    agg_split = _aggregate_stage(y_split, src, dst, edge_weight, zeros)
    z = _z_stage(x, W_root)
    out = _combine_stage(agg_split, z, s.reshape(1, D), b2.reshape(1, D))
    return out
